# Initial kernel scaffold; baseline (speedup 1.0000x reference)
import jax
import jax.numpy as jnp
from jax.experimental import pallas as pl


def kernel(z, slab):
    raise NotImplementedError("write your pallas kernel here")


# speedup vs baseline: 7.1061x; 7.1061x over previous
import functools

import jax
import jax.numpy as jnp
from jax.experimental import pallas as pl
from jax.experimental.pallas import tpu as pltpu

EPS = 1e-5
LANES = 128
_TILE = 4096


def _h1(z, v_ref):
    return jnp.maximum(
        z[:, 0:1] * v_ref[0:1, :] + z[:, 1:2] * v_ref[1:2, :] + v_ref[2:3, :],
        0.0)


def _stats1_kernel(z_ref, v_ref, s_ref):
    h = _h1(z_ref[...], v_ref)
    s_ref[0:1, 0:1, :] = jnp.sum(h, axis=0, keepdims=True).reshape(1, 1, LANES)
    s_ref[0:1, 1:2, :] = jnp.sum(h * h, axis=0, keepdims=True).reshape(1, 1, LANES)


def _stats2_kernel(z_ref, v_ref, w2_ref, s_ref):
    h = _h1(z_ref[...], v_ref)
    h = jnp.maximum(
        jnp.dot(h, w2_ref[...], preferred_element_type=jnp.float32)
        + v_ref[3:4, :], 0.0)
    s_ref[0:1, 0:1, :] = jnp.sum(h, axis=0, keepdims=True).reshape(1, 1, LANES)
    s_ref[0:1, 1:2, :] = jnp.sum(h * h, axis=0, keepdims=True).reshape(1, 1, LANES)


def _out_kernel(z_ref, v_ref, w2_ref, w3_ref, o_ref):
    h = _h1(z_ref[...], v_ref)
    h = jnp.maximum(
        jnp.dot(h, w2_ref[...], preferred_element_type=jnp.float32)
        + v_ref[3:4, :], 0.0)
    h = jnp.dot(h, w3_ref[...], preferred_element_type=jnp.float32) + v_ref[4:5, :]
    o_ref[...] = jax.nn.sigmoid(h)


def _bn_fold(stats, inv_b, g, be, w, b):
    st = jnp.sum(stats, axis=0)
    m = st[0] * inv_b
    var = jnp.maximum(st[1] * inv_b - m * m, 0.0)
    scale = g[0] * jax.lax.rsqrt(var + EPS)
    shift = be[0] - m * scale
    return scale[:, None] * w, shift @ w + b[0]


def kernel(z, slab):
    L = 2
    r2 = 16
    r3 = 144
    B = z.shape[0]
    tile = _TILE if B % _TILE == 0 else 8
    T = B // tile
    inv_b = 1.0 / B

    w1a, w1b = slab[0:1], slab[1:2]
    b1 = slab[L + 0:L + 1]
    g1, be1 = slab[L + 1:L + 2], slab[L + 2:L + 3]
    b2 = slab[L + 3:L + 4]
    g2, be2 = slab[L + 4:L + 5], slab[L + 5:L + 6]
    b3 = slab[L + 6:L + 7]
    w2 = jax.lax.slice(slab, (r2, 0), (r2 + LANES, LANES))
    w3 = jax.lax.slice(slab, (r3, 0), (r3 + LANES, LANES))

    zero = jnp.zeros((1, LANES), jnp.float32)
    parallel = pltpu.CompilerParams(dimension_semantics=("parallel",))
    z_spec = pl.BlockSpec((tile, 2), lambda t: (t, 0))
    vec_spec = pl.BlockSpec((8, LANES), lambda t: (0, 0))
    w_spec = pl.BlockSpec((LANES, LANES), lambda t: (0, 0))
    stat_spec = pl.BlockSpec((1, 2, LANES), lambda t: (t, 0, 0))
    stat_shape = jax.ShapeDtypeStruct((T, 2, LANES), jnp.float32)

    vecs1 = jnp.concatenate([w1a, w1b, b1, zero, zero, zero, zero, zero])
    s1 = pl.pallas_call(
        _stats1_kernel,
        grid=(T,),
        out_shape=stat_shape,
        in_specs=[z_spec, vec_spec],
        out_specs=stat_spec,
        compiler_params=parallel,
    )(z, vecs1)

    w2f, b2f = _bn_fold(s1, inv_b, g1, be1, w2, b2)

    vecs2 = jnp.concatenate([w1a, w1b, b1, b2f[None, :], zero, zero, zero, zero])
    s2 = pl.pallas_call(
        _stats2_kernel,
        grid=(T,),
        out_shape=stat_shape,
        in_specs=[z_spec, vec_spec, w_spec],
        out_specs=stat_spec,
        compiler_params=parallel,
    )(z, vecs2, w2f)

    w3f, b3f = _bn_fold(s2, inv_b, g2, be2, w3, b3)

    vecs3 = jnp.concatenate([w1a, w1b, b1, b2f[None, :], b3f[None, :],
                             zero, zero, zero])
    out = pl.pallas_call(
        _out_kernel,
        grid=(T,),
        out_shape=jax.ShapeDtypeStruct((B, LANES), jnp.float32),
        in_specs=[z_spec, vec_spec, w_spec, w_spec],
        out_specs=pl.BlockSpec((tile, LANES), lambda t: (t, 0)),
        compiler_params=parallel,
    )(z, vecs3, w2f, w3f)
    return out


# speedup vs baseline: 9.8646x; 1.3882x over previous
import functools

import jax
import jax.numpy as jnp
from jax.experimental import pallas as pl
from jax.experimental.pallas import tpu as pltpu

EPS = 1e-5
LANES = 128


def _lane_fold(h, nb):
    acc = h[:, 0:LANES]
    for j in range(1, nb // LANES):
        acc = acc + h[:, j * LANES:(j + 1) * LANES]
    return acc


def _h1t(zt, w1t_ref, b1c_ref):
    h = jnp.dot(w1t_ref[...], zt, preferred_element_type=jnp.float32)
    return jnp.maximum(h + b1c_ref[...], 0.0)


def _h2t(h1t, w2t_ref, b2c_ref):
    h = jnp.dot(w2t_ref[...], h1t, preferred_element_type=jnp.float32)
    return jnp.maximum(h + b2c_ref[...], 0.0)


def _stats1_kernel(zt_ref, w1t_ref, b1c_ref, s_ref, *, nb):
    h = _h1t(zt_ref[...], w1t_ref, b1c_ref)
    s_ref[0, 0] = _lane_fold(h, nb)
    s_ref[0, 1] = _lane_fold(h * h, nb)


def _stats2_kernel(zt_ref, w1t_ref, b1c_ref, w2t_ref, b2c_ref, s_ref, *, nb):
    h = _h2t(_h1t(zt_ref[...], w1t_ref, b1c_ref), w2t_ref, b2c_ref)
    s_ref[0, 0] = _lane_fold(h, nb)
    s_ref[0, 1] = _lane_fold(h * h, nb)


def _out_kernel(zt_ref, w1t_ref, b1c_ref, w2t_ref, b2c_ref, w3_ref, b3_ref,
                o_ref, *, nb):
    h2t = _h2t(_h1t(zt_ref[...], w1t_ref, b1c_ref), w2t_ref, b2c_ref)
    w3 = w3_ref[...]
    b3 = b3_ref[...]
    for j in range(nb // 256):
        blk = jax.lax.dot_general(
            h2t[:, j * 256:(j + 1) * 256], w3,
            dimension_numbers=(((0,), (0,)), ((), ())),
            preferred_element_type=jnp.float32)
        o_ref[j * 256:(j + 1) * 256, :] = jax.nn.sigmoid(blk + b3)


def _bn_fold(stats, inv_b, g, be, w, b):
    st = jnp.sum(stats, axis=(0, 3))
    m = st[0] * inv_b
    var = jnp.maximum(st[1] * inv_b - m * m, 0.0)
    scale = g * jax.lax.rsqrt(var + EPS)
    shift = be - m * scale
    return scale[:, None] * w, shift @ w + b


def kernel(z, slab):
    r2, r3 = 16, 144
    d2, d1, d0 = 32, 64, 128
    B = z.shape[0]
    nb = 2048
    while B % nb:
        nb //= 2
    T = B // nb
    inv_b = 1.0 / B

    w1t = jnp.transpose(jax.lax.slice(slab, (0, 0), (2, d2)))
    b1c = jax.lax.slice(slab, (2, 0), (3, d2)).T
    g1, be1 = slab[3, :d2], slab[4, :d2]
    b2 = slab[5, :d1]
    g2, be2 = slab[6, :d1], slab[7, :d1]
    b3 = slab[8:9, :]
    w2 = jax.lax.slice(slab, (r2, 0), (r2 + d2, d1))
    w3 = jax.lax.slice(slab, (r3, 0), (r3 + d1, d0))

    zt = jnp.transpose(z)

    parallel = pltpu.CompilerParams(dimension_semantics=("parallel",))
    zt_spec = pl.BlockSpec((2, nb), lambda t: (0, t))
    small = lambda a: pl.BlockSpec(a.shape, lambda t: (0,) * a.ndim)

    s1 = pl.pallas_call(
        functools.partial(_stats1_kernel, nb=nb),
        grid=(T,),
        out_shape=jax.ShapeDtypeStruct((T, 2, d2, LANES), jnp.float32),
        in_specs=[zt_spec, small(w1t), small(b1c)],
        out_specs=pl.BlockSpec((1, 2, d2, LANES), lambda t: (t, 0, 0, 0)),
        compiler_params=parallel,
    )(zt, w1t, b1c)

    w2f, b2f = _bn_fold(s1, inv_b, g1, be1, w2, b2)
    w2t = jnp.transpose(w2f)
    b2c = b2f[:, None]

    s2 = pl.pallas_call(
        functools.partial(_stats2_kernel, nb=nb),
        grid=(T,),
        out_shape=jax.ShapeDtypeStruct((T, 2, d1, LANES), jnp.float32),
        in_specs=[zt_spec, small(w1t), small(b1c), small(w2t), small(b2c)],
        out_specs=pl.BlockSpec((1, 2, d1, LANES), lambda t: (t, 0, 0, 0)),
        compiler_params=parallel,
    )(zt, w1t, b1c, w2t, b2c)

    w3f, b3f = _bn_fold(s2, inv_b, g2, be2, w3, b3[0])
    b3r = b3f[None, :]

    out = pl.pallas_call(
        functools.partial(_out_kernel, nb=nb),
        grid=(T,),
        out_shape=jax.ShapeDtypeStruct((B, LANES), jnp.float32),
        in_specs=[zt_spec, small(w1t), small(b1c), small(w2t), small(b2c),
                  small(w3f), small(b3r)],
        out_specs=pl.BlockSpec((nb, LANES), lambda t: (t, 0)),
        compiler_params=parallel,
    )(zt, w1t, b1c, w2t, b2c, w3f, b3r)
    return out


# speedup vs baseline: 24.4645x; 2.4800x over previous
import functools

import jax
import jax.numpy as jnp
from jax.experimental import pallas as pl
from jax.experimental.pallas import tpu as pltpu

EPS = 1e-5
LANES = 128
_NB = 8192


def _lane_fold(h, nb):
    acc = h[:, 0:LANES]
    for j in range(1, nb // LANES):
        acc = acc + h[:, j * LANES:(j + 1) * LANES]
    return acc


def _h1t(zt, w1t_ref, b1c_ref):
    h = jnp.dot(w1t_ref[...], zt, preferred_element_type=jnp.float32)
    return jnp.maximum(h + b1c_ref[...], 0.0)


def _h2t(h1t, w2t_ref, b2c_ref):
    h = jnp.dot(w2t_ref[...], h1t, preferred_element_type=jnp.float32)
    return jnp.maximum(h + b2c_ref[...], 0.0)


def _accum_stats(s_ref, h, nb):
    @pl.when(pl.program_id(0) == 0)
    def _():
        s_ref[...] = jnp.zeros_like(s_ref)

    s_ref[0] += _lane_fold(h, nb)
    s_ref[1] += _lane_fold(h * h, nb)


def _stats1_kernel(zt_ref, w1t_ref, b1c_ref, s_ref, *, nb):
    _accum_stats(s_ref, _h1t(zt_ref[...], w1t_ref, b1c_ref), nb)


def _stats2_kernel(zt_ref, w1t_ref, b1c_ref, w2t_ref, b2c_ref, s_ref, *, nb):
    _accum_stats(
        s_ref, _h2t(_h1t(zt_ref[...], w1t_ref, b1c_ref), w2t_ref, b2c_ref), nb)


def _out_kernel(zt_ref, w1t_ref, b1c_ref, w2t_ref, b2c_ref, w3d_ref, b3d_ref,
                o_ref, *, nb):
    h2t = _h2t(_h1t(zt_ref[...], w1t_ref, b1c_ref), w2t_ref, b2c_ref)
    w3d = w3d_ref[...]
    b3d = b3d_ref[...]
    for j in range(nb // 512):
        lo, hi = j * 512, j * 512 + 256
        pair = jnp.concatenate(
            [h2t[:, lo:hi], h2t[:, hi:hi + 256]], axis=0)
        blk = jax.lax.dot_general(
            pair, w3d, dimension_numbers=(((0,), (0,)), ((), ())),
            preferred_element_type=jnp.float32)
        blk = jax.nn.sigmoid(blk + b3d)
        o_ref[lo:hi, :] = blk[:, 0:LANES]
        o_ref[hi:hi + 256, :] = blk[:, LANES:2 * LANES]


def _bn_fold(stats, inv_b, g, be, w, b):
    st = jnp.sum(stats, axis=2)
    m = st[0] * inv_b
    var = jnp.maximum(st[1] * inv_b - m * m, 0.0)
    scale = g * jax.lax.rsqrt(var + EPS)
    shift = be - m * scale
    return scale[:, None] * w, shift @ w + b


def kernel(z, slab):
    r2, r3 = 16, 144
    d2, d1, d0 = 32, 64, 128
    B = z.shape[0]
    nb = _NB
    while B % nb:
        nb //= 2
    T = B // nb
    inv_b = 1.0 / B

    w1t = jnp.transpose(jax.lax.slice(slab, (0, 0), (2, d2)))
    b1c = jax.lax.slice(slab, (2, 0), (3, d2)).T
    g1, be1 = slab[3, :d2], slab[4, :d2]
    b2 = slab[5, :d1]
    g2, be2 = slab[6, :d1], slab[7, :d1]
    b3 = slab[8, :]
    w2 = jax.lax.slice(slab, (r2, 0), (r2 + d2, d1))
    w3 = jax.lax.slice(slab, (r3, 0), (r3 + d1, d0))

    zt = jnp.transpose(z)

    arb = pltpu.CompilerParams(dimension_semantics=("arbitrary",))
    zt_spec = pl.BlockSpec((2, nb), lambda t: (0, t))
    small = lambda a: pl.BlockSpec(a.shape, lambda t: (0,) * a.ndim)

    s1 = pl.pallas_call(
        functools.partial(_stats1_kernel, nb=nb),
        grid=(T,),
        out_shape=jax.ShapeDtypeStruct((2, d2, LANES), jnp.float32),
        in_specs=[zt_spec, small(w1t), small(b1c)],
        out_specs=pl.BlockSpec((2, d2, LANES), lambda t: (0, 0, 0)),
        compiler_params=arb,
    )(zt, w1t, b1c)

    w2f, b2f = _bn_fold(s1, inv_b, g1, be1, w2, b2)
    w2t = jnp.transpose(w2f)
    b2c = b2f[:, None]

    s2 = pl.pallas_call(
        functools.partial(_stats2_kernel, nb=nb),
        grid=(T,),
        out_shape=jax.ShapeDtypeStruct((2, d1, LANES), jnp.float32),
        in_specs=[zt_spec, small(w1t), small(b1c), small(w2t), small(b2c)],
        out_specs=pl.BlockSpec((2, d1, LANES), lambda t: (0, 0, 0)),
        compiler_params=arb,
    )(zt, w1t, b1c, w2t, b2c)

    w3f, b3f = _bn_fold(s2, inv_b, g2, be2, w3, b3)
    w3d = jnp.zeros((2 * d1, 2 * d0), jnp.float32)
    w3d = w3d.at[:d1, :d0].set(w3f).at[d1:, d0:].set(w3f)
    b3d = jnp.concatenate([b3f, b3f])[None, :]

    out = pl.pallas_call(
        functools.partial(_out_kernel, nb=nb),
        grid=(T,),
        out_shape=jax.ShapeDtypeStruct((B, LANES), jnp.float32),
        in_specs=[zt_spec, small(w1t), small(b1c), small(w2t), small(b2c),
                  small(w3d), small(b3d)],
        out_specs=pl.BlockSpec((nb, LANES), lambda t: (t, 0)),
        compiler_params=arb,
    )(zt, w1t, b1c, w2t, b2c, w3d, b3d)
    return out


# speedup vs baseline: 30.6819x; 1.2541x over previous
import functools

import jax
import jax.numpy as jnp
from jax.experimental import pallas as pl
from jax.experimental.pallas import tpu as pltpu

EPS = 1e-5
LANES = 128
_NB = 32768


def _lane_fold(h, nb):
    acc = h[:, 0:LANES]
    for j in range(1, nb // LANES):
        acc = acc + h[:, j * LANES:(j + 1) * LANES]
    return acc


def _h1t(zt_ref, w1p_ref):
    z0 = zt_ref[0:1, :]
    z1 = zt_ref[1:2, :]
    h = w1p_ref[:, 0:1] * z0 + w1p_ref[:, 1:2] * z1 + w1p_ref[:, 2:3]
    return jnp.maximum(h, 0.0)


def _h2t(h1t, w2t_ref, b2c_ref):
    h = jnp.dot(w2t_ref[...], h1t, preferred_element_type=jnp.float32)
    return jnp.maximum(h + b2c_ref[...], 0.0)


def _accum_stats(s_ref, h, nb):
    @pl.when(pl.program_id(0) == 0)
    def _():
        s_ref[...] = jnp.zeros_like(s_ref)

    s_ref[0] += _lane_fold(h, nb)
    s_ref[1] += _lane_fold(h * h, nb)


def _stats1_kernel(zt_ref, w1t_ref, b1c_ref, s_ref, *, nb):
    h = jnp.dot(w1t_ref[...], zt_ref[...], preferred_element_type=jnp.float32)
    h = jnp.maximum(h + b1c_ref[...], 0.0)
    _accum_stats(s_ref, h, nb)


def _stats2_kernel(zt_ref, w1p_ref, w2t_ref, b2c_ref, s_ref, h2_ref, *, nb):
    h2 = _h2t(_h1t(zt_ref, w1p_ref), w2t_ref, b2c_ref)
    _accum_stats(s_ref, h2, nb)
    h2_ref[...] = h2.astype(jnp.bfloat16)


def _out_kernel(h2_ref, w3d_ref, b3d_ref, o_ref, *, nb):
    w3d = w3d_ref[...]
    b3d = b3d_ref[...]
    for j in range(nb // 512):
        lo, hi = j * 512, j * 512 + 256
        pair = jnp.concatenate(
            [h2_ref[:, lo:hi], h2_ref[:, hi:hi + 256]], axis=0)
        blk = jax.lax.dot_general(
            pair, w3d, dimension_numbers=(((0,), (0,)), ((), ())),
            preferred_element_type=jnp.float32)
        blk = jax.nn.sigmoid(blk + b3d)
        o_ref[lo:hi, :] = blk[:, 0:LANES]
        o_ref[hi:hi + 256, :] = blk[:, LANES:2 * LANES]


def _bn_fold(stats, inv_b, g, be, w, b):
    st = jnp.sum(stats, axis=2)
    m = st[0] * inv_b
    var = jnp.maximum(st[1] * inv_b - m * m, 0.0)
    scale = g * jax.lax.rsqrt(var + EPS)
    shift = be - m * scale
    return scale[:, None] * w, shift @ w + b


def kernel(z, slab):
    r2, r3 = 16, 144
    d2, d1, d0 = 32, 64, 128
    B = z.shape[0]
    nb = _NB
    while B % nb:
        nb //= 2
    T = B // nb
    inv_b = 1.0 / B

    w1p = jnp.transpose(jax.lax.slice(slab, (0, 0), (3, d2)))
    w1t = jax.lax.slice(w1p, (0, 0), (d2, 2))
    b1c = jax.lax.slice(w1p, (0, 2), (d2, 3))
    g1, be1 = slab[3, :d2], slab[4, :d2]
    b2 = slab[5, :d1]
    g2, be2 = slab[6, :d1], slab[7, :d1]
    b3 = slab[8, :]
    w2 = jax.lax.slice(slab, (r2, 0), (r2 + d2, d1))
    w3 = jax.lax.slice(slab, (r3, 0), (r3 + d1, d0))

    zt = jnp.transpose(z)

    arb = pltpu.CompilerParams(dimension_semantics=("arbitrary",))
    zt_spec = pl.BlockSpec((2, nb), lambda t: (0, t))
    h2_spec = pl.BlockSpec((d1, nb), lambda t: (0, t))
    small = lambda a: pl.BlockSpec(a.shape, lambda t: (0,) * a.ndim)

    s1 = pl.pallas_call(
        functools.partial(_stats1_kernel, nb=nb),
        grid=(T,),
        out_shape=jax.ShapeDtypeStruct((2, d2, LANES), jnp.float32),
        in_specs=[zt_spec, small(w1t), small(b1c)],
        out_specs=pl.BlockSpec((2, d2, LANES), lambda t: (0, 0, 0)),
        compiler_params=arb,
    )(zt, w1t, b1c)

    w2f, b2f = _bn_fold(s1, inv_b, g1, be1, w2, b2)
    w2t = jnp.transpose(w2f)
    b2c = b2f[:, None]

    s2, h2t = pl.pallas_call(
        functools.partial(_stats2_kernel, nb=nb),
        grid=(T,),
        out_shape=(jax.ShapeDtypeStruct((2, d1, LANES), jnp.float32),
                   jax.ShapeDtypeStruct((d1, B), jnp.bfloat16)),
        in_specs=[zt_spec, small(w1p), small(w2t), small(b2c)],
        out_specs=(pl.BlockSpec((2, d1, LANES), lambda t: (0, 0, 0)), h2_spec),
        compiler_params=arb,
    )(zt, w1p, w2t, b2c)

    w3f, b3f = _bn_fold(s2, inv_b, g2, be2, w3, b3)
    w3d = jnp.zeros((2 * d1, 2 * d0), jnp.float32)
    w3d = w3d.at[:d1, :d0].set(w3f).at[d1:, d0:].set(w3f)
    w3d = w3d.astype(jnp.bfloat16)
    b3d = jnp.concatenate([b3f, b3f])[None, :]

    out = pl.pallas_call(
        functools.partial(_out_kernel, nb=nb),
        grid=(T,),
        out_shape=jax.ShapeDtypeStruct((B, LANES), jnp.float32),
        in_specs=[h2_spec, small(w3d), small(b3d)],
        out_specs=pl.BlockSpec((nb, LANES), lambda t: (t, 0)),
        compiler_params=arb,
    )(h2t, w3d, b3d)
    return out
